# baseline (speedup 1.0000x reference)
import functools

import jax
import jax.numpy as jnp
from jax import lax
from jax.experimental import pallas as pl
from jax.experimental.pallas import tpu as pltpu
from jax.experimental.pallas import tpu_sc as plsc

_B, _N, _C, _H = 8, 577, 768, 12
_Dh = _C // _H
_SCALE = _Dh ** -0.5
_KEEP = max(1, int((_N - 1) * 0.5))
_NP = 640
_LANES = 16
_NCHUNK = _NP // _LANES
_I32_MIN = -2147483648
_I32_MAX = 2147483647
_NEG = -1e30


_NS = 584


def _stats_body(x_ref, w_ref, b_ref, expl_ref,
                q_ref, k_ref, v_ref, gucb_ref, sc_ref):
    h = pl.program_id(1)
    xb = x_ref[0]
    cdims = (((1,), (1,)), ((), ()))
    qkv = lax.dot_general(xb, w_ref[0], cdims,
                          preferred_element_type=jnp.float32) + b_ref[0, 0]
    q = qkv[:, 0:_Dh]
    k = qkv[:, _Dh:2 * _Dh]
    v = qkv[:, 2 * _Dh:3 * _Dh]
    vrow = lax.broadcasted_iota(jnp.int32, (_NP, _Dh), 0)
    v = jnp.where(vrow < _N, v, 0.0)
    q_ref[0, 0] = q[:_NS]
    k_ref[0, 0] = k
    v_ref[0, 0] = v.astype(jnp.bfloat16)

    s = lax.dot_general(q[:_NS], k, cdims,
                        preferred_element_type=jnp.float32) * _SCALE
    colid = lax.broadcasted_iota(jnp.int32, (_NS, _NP), 1)
    s = jnp.where(colid < _N, s, _NEG)
    mx = jnp.max(s, axis=1, keepdims=True)
    p = jnp.exp(s - mx)
    probs = p / jnp.sum(p, axis=1, keepdims=True)
    rowid = lax.broadcasted_iota(jnp.int32, (_NS, _NP), 0)
    probs = jnp.where(rowid < _N, probs, 0.0)
    patch = jnp.sum(probs, axis=0) / jnp.float32(_N)
    sc = patch + expl_ref[0, 0]
    for i in range(_H):
        @pl.when(h == i)
        def _(i=i):
            sc_ref[i] = sc

    @pl.when(h == _H - 1)
    def _():
        gucb_ref[0, 0] = jnp.sum(sc_ref[...], axis=0) / jnp.float32(_H)


def _sc_mask_body(gucb_hbm, mask_hbm, row_v, keys_v, mask_v):
    wid = lax.axis_index("s") * 2 + lax.axis_index("c")

    @pl.when(wid < _B)
    def _():
        pltpu.sync_copy(gucb_hbm.at[wid, 0], row_v)
        for c in range(_NCHUNK):
            bits = lax.bitcast_convert_type(row_v[pl.ds(c * _LANES, _LANES)],
                                            jnp.int32)
            key = jnp.where(bits >= 0, bits, _I32_MIN - bits)
            lane = lax.iota(jnp.int32, 16) + (c * _LANES)
            valid = (lane >= 1) & (lane <= _N - 1)
            keys_v[pl.ds(c * _LANES, _LANES)] = jnp.where(valid, key, _I32_MIN)

        def count_gt(t):
            acc = jnp.zeros((16,), jnp.int32)
            for c in range(_NCHUNK):
                kc = keys_v[pl.ds(c * _LANES, _LANES)]
                acc = acc + plsc.all_reduce_population_count(kc > t)
            return acc

        def bs_body(_, carry):
            lo, hi = carry
            mid = (lo & hi) + ((lo ^ hi) >> 1)
            take = lo < hi
            p = count_gt(mid) < _KEEP
            lo2 = jnp.where(p, lo, mid + 1)
            hi2 = jnp.where(p, mid, hi)
            return (jnp.where(take, lo2, lo), jnp.where(take, hi2, hi))

        lo, _hi = lax.fori_loop(0, 32, bs_body,
                                (jnp.full((16,), _I32_MIN, jnp.int32),
                                 jnp.full((16,), _I32_MAX, jnp.int32)))
        t = lo
        nties = _KEEP - count_gt(t)

        def cnt_eq_le(m):
            acc = jnp.zeros((16,), jnp.int32)
            for c in range(_NCHUNK):
                kc = keys_v[pl.ds(c * _LANES, _LANES)]
                lane = lax.iota(jnp.int32, 16) + (c * _LANES)
                acc = acc + plsc.all_reduce_population_count(
                    (kc == t) & (lane <= m))
            return acc

        def bs2_body(_, carry):
            lo2, hi2 = carry
            mid = (lo2 + hi2) >> 1
            take = lo2 < hi2
            p = cnt_eq_le(mid) >= nties
            nlo = jnp.where(p, lo2, mid + 1)
            nhi = jnp.where(p, mid, hi2)
            return (jnp.where(take, nlo, lo2), jnp.where(take, nhi, hi2))

        m0, _m1 = lax.fori_loop(0, 10, bs2_body,
                                (jnp.zeros((16,), jnp.int32),
                                 jnp.full((16,), _NP - 1, jnp.int32)))
        mstar = jnp.where(nties > 0, m0, jnp.full((16,), -1, jnp.int32))

        for c in range(_NCHUNK):
            kc = keys_v[pl.ds(c * _LANES, _LANES)]
            lane = lax.iota(jnp.int32, 16) + (c * _LANES)
            keep = (kc > t) | ((kc == t) & (lane <= mstar))
            mvec = jnp.where(keep, 1.0, 0.0)
            if c == 0:
                mvec = jnp.where(lax.iota(jnp.int32, 16) == 0, 1.0, mvec)
            mask_v[pl.ds(c * _LANES, _LANES)] = mvec
        pltpu.sync_copy(mask_v, mask_hbm.at[wid, 0])


def _attn_body(q_ref, k_ref, v_ref, m_ref, wp_ref, bp_ref, out_ref, ctx_ref):
    h = pl.program_id(1)
    cdims = (((1,), (1,)), ((), ()))
    s = lax.dot_general(q_ref[0, 0], k_ref[0, 0], cdims,
                        preferred_element_type=jnp.float32) * _SCALE
    colid = lax.broadcasted_iota(jnp.int32, (_NS, _NP), 1)
    s = jnp.where(colid < _N, s, _NEG)
    mx = jnp.max(s, axis=1, keepdims=True)
    p = jnp.exp(s - mx)
    srow = jnp.sum(p, axis=1, keepdims=True)
    m = m_ref[0, 0]
    pm = p * jnp.maximum(m[:_NS, None], m[None, :])
    denom = jnp.sum(pm, axis=1, keepdims=True) + 1e-8 * srow
    ctx = lax.dot_general(pm.astype(jnp.bfloat16), v_ref[0, 0],
                          (((1,), (0,)), ((), ())),
                          preferred_element_type=jnp.float32)
    ctx = ctx / denom
    for i in range(_H):
        @pl.when(h == i)
        def _(i=i):
            ctx_ref[:, i * _Dh:(i + 1) * _Dh] = ctx.astype(jnp.bfloat16)

    @pl.when(h == _H - 1)
    def _():
        out_ref[0] = lax.dot_general(
            ctx_ref[...], wp_ref[0], cdims,
            preferred_element_type=jnp.float32) + bp_ref[0, 0]


def kernel(x, Wqkv, bqkv, Wproj, bproj, count_score_buffer, iteration):
    f32 = jnp.float32
    Wq = Wqkv[0 * _C:1 * _C].reshape(_H, _Dh, _C)
    Wk = Wqkv[1 * _C:2 * _C].reshape(_H, _Dh, _C)
    Wv = Wqkv[2 * _C:3 * _C].reshape(_H, _Dh, _C)
    Wcat = jnp.concatenate([Wq, Wk, Wv], axis=1)
    bq = bqkv[0 * _C:1 * _C].reshape(_H, 1, _Dh)
    bk = bqkv[1 * _C:2 * _C].reshape(_H, 1, _Dh)
    bv = bqkv[2 * _C:3 * _C].reshape(_H, 1, _Dh)
    bcat = jnp.concatenate([bq, bk, bv], axis=2)
    ucb_expl = jnp.sqrt(jnp.log(jnp.asarray(iteration + 1.0, dtype=f32))
                        / (count_score_buffer[:, 1:] + 1e-06))
    expl = jnp.zeros((_H, _NP), f32).at[:, 1:_N].set(ucb_expl)
    expl = expl.reshape(_H, 1, _NP)
    Wp = Wproj.reshape(1, _C, _C).astype(jnp.bfloat16)
    bp = bproj.reshape(1, 1, _C)

    k_shape = jax.ShapeDtypeStruct((_B, _H, _NP, _Dh), f32)
    v_shape = jax.ShapeDtypeStruct((_B, _H, _NP, _Dh), jnp.bfloat16)
    q_shape = jax.ShapeDtypeStruct((_B, _H, _NS, _Dh), f32)
    q, k, v, gucb = pl.pallas_call(
        _stats_body,
        grid=(_B, _H),
        in_specs=[
            pl.BlockSpec((1, _NP, _C), lambda b, h: (b, 0, 0)),
            pl.BlockSpec((1, 3 * _Dh, _C), lambda b, h: (h, 0, 0)),
            pl.BlockSpec((1, 1, 3 * _Dh), lambda b, h: (h, 0, 0)),
            pl.BlockSpec((1, 1, _NP), lambda b, h: (h, 0, 0)),
        ],
        out_specs=[
            pl.BlockSpec((1, 1, _NS, _Dh), lambda b, h: (b, h, 0, 0)),
            pl.BlockSpec((1, 1, _NP, _Dh), lambda b, h: (b, h, 0, 0)),
            pl.BlockSpec((1, 1, _NP, _Dh), lambda b, h: (b, h, 0, 0)),
            pl.BlockSpec((1, 1, _NP), lambda b, h: (b, 0, 0)),
        ],
        out_shape=[q_shape, k_shape, v_shape,
                   jax.ShapeDtypeStruct((_B, 1, _NP), f32)],
        scratch_shapes=[pltpu.VMEM((_H, _NP), f32)],
    )(x, Wcat, bcat, expl)

    mask = pl.kernel(
        _sc_mask_body,
        mesh=plsc.VectorSubcoreMesh(core_axis_name="c", subcore_axis_name="s"),
        compiler_params=pltpu.CompilerParams(needs_layout_passes=False),
        out_type=jax.ShapeDtypeStruct((_B, 1, _NP), f32),
        scratch_types=[
            pltpu.VMEM((_NP,), f32),
            pltpu.VMEM((_NP,), jnp.int32),
            pltpu.VMEM((_NP,), f32),
        ],
    )(gucb)

    out_p = pl.pallas_call(
        _attn_body,
        grid=(_B, _H),
        in_specs=[
            pl.BlockSpec((1, 1, _NS, _Dh), lambda b, h: (b, h, 0, 0)),
            pl.BlockSpec((1, 1, _NP, _Dh), lambda b, h: (b, h, 0, 0)),
            pl.BlockSpec((1, 1, _NP, _Dh), lambda b, h: (b, h, 0, 0)),
            pl.BlockSpec((1, 1, _NP), lambda b, h: (b, 0, 0)),
            pl.BlockSpec((1, _C, _C), lambda b, h: (0, 0, 0)),
            pl.BlockSpec((1, 1, _C), lambda b, h: (0, 0, 0)),
        ],
        out_specs=pl.BlockSpec((1, _NS, _C), lambda b, h: (b, 0, 0)),
        out_shape=jax.ShapeDtypeStruct((_B, _NS, _C), f32),
        scratch_shapes=[pltpu.VMEM((_NS, _C), jnp.bfloat16)],
    )(q, k, v, mask, Wp, bp)

    return out_p[:, :_N, :]


# speedup vs baseline: 1.0298x; 1.0298x over previous
import functools

import jax
import jax.numpy as jnp
from jax import lax
from jax.experimental import pallas as pl
from jax.experimental.pallas import tpu as pltpu
from jax.experimental.pallas import tpu_sc as plsc

_B, _N, _C, _H = 8, 577, 768, 12
_Dh = _C // _H
_SCALE = _Dh ** -0.5
_KEEP = max(1, int((_N - 1) * 0.5))
_NP = 640
_LANES = 16
_NCHUNK = _NP // _LANES
_I32_MIN = -2147483648
_I32_MAX = 2147483647
_NEG = -1e30


_NS = 584


def _stats_body(x_ref, w_ref, b_ref, expl_ref,
                q_ref, k_ref, v_ref, gucb_ref, sc_ref):
    h = pl.program_id(1)
    xb = x_ref[0]
    cdims = (((1,), (1,)), ((), ()))
    qkv = lax.dot_general(xb, w_ref[0], cdims,
                          preferred_element_type=jnp.float32) + b_ref[0, 0]
    q = qkv[:, 0:_Dh]
    k = qkv[:, _Dh:2 * _Dh]
    v = qkv[:, 2 * _Dh:3 * _Dh]
    vrow = lax.broadcasted_iota(jnp.int32, (_NP, _Dh), 0)
    v = jnp.where(vrow < _N, v, 0.0)
    q_ref[0, 0] = q[:_NS]
    k_ref[0, 0] = k
    v_ref[0, 0] = v

    s = lax.dot_general(q[:_NS], k, cdims,
                        preferred_element_type=jnp.float32) * _SCALE
    colid = lax.broadcasted_iota(jnp.int32, (_NS, _NP), 1)
    s = jnp.where(colid < _N, s, _NEG)
    mx = jnp.max(s, axis=1, keepdims=True)
    p = jnp.exp(s - mx)
    probs = p / jnp.sum(p, axis=1, keepdims=True)
    rowid = lax.broadcasted_iota(jnp.int32, (_NS, _NP), 0)
    probs = jnp.where(rowid < _N, probs, 0.0)
    patch = jnp.sum(probs, axis=0) / jnp.float32(_N)
    sc = patch + expl_ref[0, 0]
    for i in range(_H):
        @pl.when(h == i)
        def _(i=i):
            sc_ref[i] = sc

    @pl.when(h == _H - 1)
    def _():
        gucb_ref[0, 0] = jnp.sum(sc_ref[...], axis=0) / jnp.float32(_H)


def _sc_mask_body(gucb_hbm, mask_hbm, row_v, keys_v, mask_v):
    wid = lax.axis_index("s") * 2 + lax.axis_index("c")

    @pl.when(wid < _B)
    def _():
        pltpu.sync_copy(gucb_hbm.at[wid, 0], row_v)
        for c in range(_NCHUNK):
            bits = lax.bitcast_convert_type(row_v[pl.ds(c * _LANES, _LANES)],
                                            jnp.int32)
            key = jnp.where(bits >= 0, bits, _I32_MIN - bits)
            lane = lax.iota(jnp.int32, 16) + (c * _LANES)
            valid = (lane >= 1) & (lane <= _N - 1)
            keys_v[pl.ds(c * _LANES, _LANES)] = jnp.where(valid, key, _I32_MIN)

        def count_gt(t):
            acc = jnp.zeros((16,), jnp.int32)
            for c in range(_NCHUNK):
                kc = keys_v[pl.ds(c * _LANES, _LANES)]
                acc = acc + plsc.all_reduce_population_count(kc > t)
            return acc

        def bs_body(_, carry):
            lo, hi = carry
            mid = (lo & hi) + ((lo ^ hi) >> 1)
            take = lo < hi
            p = count_gt(mid) < _KEEP
            lo2 = jnp.where(p, lo, mid + 1)
            hi2 = jnp.where(p, mid, hi)
            return (jnp.where(take, lo2, lo), jnp.where(take, hi2, hi))

        lo, _hi = lax.fori_loop(0, 32, bs_body,
                                (jnp.full((16,), _I32_MIN, jnp.int32),
                                 jnp.full((16,), _I32_MAX, jnp.int32)))
        t = lo
        nties = _KEEP - count_gt(t)

        def cnt_eq_le(m):
            acc = jnp.zeros((16,), jnp.int32)
            for c in range(_NCHUNK):
                kc = keys_v[pl.ds(c * _LANES, _LANES)]
                lane = lax.iota(jnp.int32, 16) + (c * _LANES)
                acc = acc + plsc.all_reduce_population_count(
                    (kc == t) & (lane <= m))
            return acc

        def bs2_body(_, carry):
            lo2, hi2 = carry
            mid = (lo2 + hi2) >> 1
            take = lo2 < hi2
            p = cnt_eq_le(mid) >= nties
            nlo = jnp.where(p, lo2, mid + 1)
            nhi = jnp.where(p, mid, hi2)
            return (jnp.where(take, nlo, lo2), jnp.where(take, nhi, hi2))

        m0, _m1 = lax.fori_loop(0, 10, bs2_body,
                                (jnp.zeros((16,), jnp.int32),
                                 jnp.full((16,), _NP - 1, jnp.int32)))
        mstar = jnp.where(nties > 0, m0, jnp.full((16,), -1, jnp.int32))

        for c in range(_NCHUNK):
            kc = keys_v[pl.ds(c * _LANES, _LANES)]
            lane = lax.iota(jnp.int32, 16) + (c * _LANES)
            keep = (kc > t) | ((kc == t) & (lane <= mstar))
            mvec = jnp.where(keep, 1.0, 0.0)
            if c == 0:
                mvec = jnp.where(lax.iota(jnp.int32, 16) == 0, 1.0, mvec)
            mask_v[pl.ds(c * _LANES, _LANES)] = mvec
        pltpu.sync_copy(mask_v, mask_hbm.at[wid, 0])


def _attn_body(q_ref, k_ref, v_ref, m_ref, wp_ref, bp_ref, out_ref, ctx_ref):
    h = pl.program_id(1)
    cdims = (((1,), (1,)), ((), ()))
    s = lax.dot_general(q_ref[0, 0], k_ref[0, 0], cdims,
                        preferred_element_type=jnp.float32) * _SCALE
    colid = lax.broadcasted_iota(jnp.int32, (_NS, _NP), 1)
    s = jnp.where(colid < _N, s, _NEG)
    mx = jnp.max(s, axis=1, keepdims=True)
    p = jnp.exp(s - mx)
    srow = jnp.sum(p, axis=1, keepdims=True)
    m = m_ref[0, 0]
    pm = p * jnp.maximum(m[:_NS, None], m[None, :])
    denom = jnp.sum(pm, axis=1, keepdims=True) + 1e-8 * srow
    ctx = lax.dot_general(pm, v_ref[0, 0], (((1,), (0,)), ((), ())),
                          preferred_element_type=jnp.float32)
    ctx = ctx / denom
    for i in range(_H):
        @pl.when(h == i)
        def _(i=i):
            ctx_ref[:, i * _Dh:(i + 1) * _Dh] = ctx

    @pl.when(h == _H - 1)
    def _():
        out_ref[0] = lax.dot_general(
            ctx_ref[...], wp_ref[0], cdims,
            preferred_element_type=jnp.float32) + bp_ref[0, 0]


def kernel(x, Wqkv, bqkv, Wproj, bproj, count_score_buffer, iteration):
    f32 = jnp.float32
    Wq = Wqkv[0 * _C:1 * _C].reshape(_H, _Dh, _C)
    Wk = Wqkv[1 * _C:2 * _C].reshape(_H, _Dh, _C)
    Wv = Wqkv[2 * _C:3 * _C].reshape(_H, _Dh, _C)
    Wcat = jnp.concatenate([Wq, Wk, Wv], axis=1)
    bq = bqkv[0 * _C:1 * _C].reshape(_H, 1, _Dh)
    bk = bqkv[1 * _C:2 * _C].reshape(_H, 1, _Dh)
    bv = bqkv[2 * _C:3 * _C].reshape(_H, 1, _Dh)
    bcat = jnp.concatenate([bq, bk, bv], axis=2)
    ucb_expl = jnp.sqrt(jnp.log(jnp.asarray(iteration + 1.0, dtype=f32))
                        / (count_score_buffer[:, 1:] + 1e-06))
    expl = jnp.zeros((_H, _NP), f32).at[:, 1:_N].set(ucb_expl)
    expl = expl.reshape(_H, 1, _NP)
    Wp = Wproj.reshape(1, _C, _C)
    bp = bproj.reshape(1, 1, _C)

    k_shape = jax.ShapeDtypeStruct((_B, _H, _NP, _Dh), f32)
    v_shape = jax.ShapeDtypeStruct((_B, _H, _NP, _Dh), f32)
    q_shape = jax.ShapeDtypeStruct((_B, _H, _NS, _Dh), f32)
    q, k, v, gucb = pl.pallas_call(
        _stats_body,
        grid=(_B, _H),
        in_specs=[
            pl.BlockSpec((1, _NP, _C), lambda b, h: (b, 0, 0)),
            pl.BlockSpec((1, 3 * _Dh, _C), lambda b, h: (h, 0, 0)),
            pl.BlockSpec((1, 1, 3 * _Dh), lambda b, h: (h, 0, 0)),
            pl.BlockSpec((1, 1, _NP), lambda b, h: (h, 0, 0)),
        ],
        out_specs=[
            pl.BlockSpec((1, 1, _NS, _Dh), lambda b, h: (b, h, 0, 0)),
            pl.BlockSpec((1, 1, _NP, _Dh), lambda b, h: (b, h, 0, 0)),
            pl.BlockSpec((1, 1, _NP, _Dh), lambda b, h: (b, h, 0, 0)),
            pl.BlockSpec((1, 1, _NP), lambda b, h: (b, 0, 0)),
        ],
        out_shape=[q_shape, k_shape, v_shape,
                   jax.ShapeDtypeStruct((_B, 1, _NP), f32)],
        scratch_shapes=[pltpu.VMEM((_H, _NP), f32)],
    )(x, Wcat, bcat, expl)

    mask = pl.kernel(
        _sc_mask_body,
        mesh=plsc.VectorSubcoreMesh(core_axis_name="c", subcore_axis_name="s"),
        compiler_params=pltpu.CompilerParams(needs_layout_passes=False),
        out_type=jax.ShapeDtypeStruct((_B, 1, _NP), f32),
        scratch_types=[
            pltpu.VMEM((_NP,), f32),
            pltpu.VMEM((_NP,), jnp.int32),
            pltpu.VMEM((_NP,), f32),
        ],
    )(gucb)

    out_p = pl.pallas_call(
        _attn_body,
        grid=(_B, _H),
        in_specs=[
            pl.BlockSpec((1, 1, _NS, _Dh), lambda b, h: (b, h, 0, 0)),
            pl.BlockSpec((1, 1, _NP, _Dh), lambda b, h: (b, h, 0, 0)),
            pl.BlockSpec((1, 1, _NP, _Dh), lambda b, h: (b, h, 0, 0)),
            pl.BlockSpec((1, 1, _NP), lambda b, h: (b, 0, 0)),
            pl.BlockSpec((1, _C, _C), lambda b, h: (0, 0, 0)),
            pl.BlockSpec((1, 1, _C), lambda b, h: (0, 0, 0)),
        ],
        out_specs=pl.BlockSpec((1, _NS, _C), lambda b, h: (b, 0, 0)),
        out_shape=jax.ShapeDtypeStruct((_B, _NS, _C), f32),
        scratch_shapes=[pltpu.VMEM((_NS, _C), f32)],
    )(q, k, v, mask, Wp, bp)

    return out_p[:, :_N, :]


# speedup vs baseline: 1.0442x; 1.0139x over previous
import functools

import jax
import jax.numpy as jnp
from jax import lax
from jax.experimental import pallas as pl
from jax.experimental.pallas import tpu as pltpu
from jax.experimental.pallas import tpu_sc as plsc

_B, _N, _C, _H = 8, 577, 768, 12
_Dh = _C // _H
_SCALE = _Dh ** -0.5
_KEEP = max(1, int((_N - 1) * 0.5))
_NP = 640
_LANES = 16
_NCHUNK = _NP // _LANES
_I32_MIN = -2147483648
_I32_MAX = 2147483647
_NEG = -1e30


_NS = 584


def _stats_body(x_ref, wq_ref, wk_ref, wv_ref, bq_ref, bk_ref, bv_ref,
                expl_ref, q_ref, k_ref, v_ref, gucb_ref, sc_ref):
    h = pl.program_id(1)
    xb = x_ref[0]
    cdims = (((1,), (1,)), ((), ()))
    q = lax.dot_general(xb, wq_ref[0], cdims,
                        preferred_element_type=jnp.float32) + bq_ref[0, 0]
    k = lax.dot_general(xb, wk_ref[0], cdims,
                        preferred_element_type=jnp.float32) + bk_ref[0, 0]
    v = lax.dot_general(xb, wv_ref[0], cdims,
                        preferred_element_type=jnp.float32) + bv_ref[0, 0]
    vrow = lax.broadcasted_iota(jnp.int32, (_NP, _Dh), 0)
    v = jnp.where(vrow < _N, v, 0.0)
    q_ref[0, 0] = q[:_NS]
    k_ref[0, 0] = k
    v_ref[0, 0] = v

    s = lax.dot_general(q[:_NS], k, cdims,
                        preferred_element_type=jnp.float32) * _SCALE
    colid = lax.broadcasted_iota(jnp.int32, (_NS, _NP), 1)
    s = jnp.where(colid < _N, s, _NEG)
    mx = jnp.max(s, axis=1, keepdims=True)
    p = jnp.exp(s - mx)
    probs = p / jnp.sum(p, axis=1, keepdims=True)
    rowid = lax.broadcasted_iota(jnp.int32, (_NS, _NP), 0)
    probs = jnp.where(rowid < _N, probs, 0.0)
    patch = jnp.sum(probs, axis=0) / jnp.float32(_N)
    sc = patch + expl_ref[0, 0]
    for i in range(_H):
        @pl.when(h == i)
        def _(i=i):
            sc_ref[i] = sc

    @pl.when(h == _H - 1)
    def _():
        gucb_ref[0, 0] = jnp.sum(sc_ref[...], axis=0) / jnp.float32(_H)


def _sc_mask_body(gucb_hbm, mask_hbm, row_v, keys_v, mask_v):
    wid = lax.axis_index("s") * 2 + lax.axis_index("c")

    @pl.when(wid < _B)
    def _():
        pltpu.sync_copy(gucb_hbm.at[wid, 0], row_v)
        for c in range(_NCHUNK):
            bits = lax.bitcast_convert_type(row_v[pl.ds(c * _LANES, _LANES)],
                                            jnp.int32)
            key = jnp.where(bits >= 0, bits, _I32_MIN - bits)
            lane = lax.iota(jnp.int32, 16) + (c * _LANES)
            valid = (lane >= 1) & (lane <= _N - 1)
            keys_v[pl.ds(c * _LANES, _LANES)] = jnp.where(valid, key, _I32_MIN)

        def count_gt(t):
            acc = jnp.zeros((16,), jnp.int32)
            for c in range(_NCHUNK):
                kc = keys_v[pl.ds(c * _LANES, _LANES)]
                acc = acc + plsc.all_reduce_population_count(kc > t)
            return acc

        def bs_body(_, carry):
            lo, hi = carry
            mid = (lo & hi) + ((lo ^ hi) >> 1)
            take = lo < hi
            p = count_gt(mid) < _KEEP
            lo2 = jnp.where(p, lo, mid + 1)
            hi2 = jnp.where(p, mid, hi)
            return (jnp.where(take, lo2, lo), jnp.where(take, hi2, hi))

        lo, _hi = lax.fori_loop(0, 32, bs_body,
                                (jnp.full((16,), _I32_MIN, jnp.int32),
                                 jnp.full((16,), _I32_MAX, jnp.int32)))
        t = lo
        nties = _KEEP - count_gt(t)

        def cnt_eq_le(m):
            acc = jnp.zeros((16,), jnp.int32)
            for c in range(_NCHUNK):
                kc = keys_v[pl.ds(c * _LANES, _LANES)]
                lane = lax.iota(jnp.int32, 16) + (c * _LANES)
                acc = acc + plsc.all_reduce_population_count(
                    (kc == t) & (lane <= m))
            return acc

        def bs2_body(_, carry):
            lo2, hi2 = carry
            mid = (lo2 + hi2) >> 1
            take = lo2 < hi2
            p = cnt_eq_le(mid) >= nties
            nlo = jnp.where(p, lo2, mid + 1)
            nhi = jnp.where(p, mid, hi2)
            return (jnp.where(take, nlo, lo2), jnp.where(take, nhi, hi2))

        m0, _m1 = lax.fori_loop(0, 10, bs2_body,
                                (jnp.zeros((16,), jnp.int32),
                                 jnp.full((16,), _NP - 1, jnp.int32)))
        mstar = jnp.where(nties > 0, m0, jnp.full((16,), -1, jnp.int32))

        for c in range(_NCHUNK):
            kc = keys_v[pl.ds(c * _LANES, _LANES)]
            lane = lax.iota(jnp.int32, 16) + (c * _LANES)
            keep = (kc > t) | ((kc == t) & (lane <= mstar))
            mvec = jnp.where(keep, 1.0, 0.0)
            if c == 0:
                mvec = jnp.where(lax.iota(jnp.int32, 16) == 0, 1.0, mvec)
            mask_v[pl.ds(c * _LANES, _LANES)] = mvec
        pltpu.sync_copy(mask_v, mask_hbm.at[wid, 0])


def _attn_body(q_ref, k_ref, v_ref, m_ref, wp_ref, bp_ref, out_ref, ctx_ref):
    h = pl.program_id(1)
    cdims = (((1,), (1,)), ((), ()))
    s = lax.dot_general(q_ref[0, 0], k_ref[0, 0], cdims,
                        preferred_element_type=jnp.float32) * _SCALE
    colid = lax.broadcasted_iota(jnp.int32, (_NS, _NP), 1)
    s = jnp.where(colid < _N, s, _NEG)
    mx = jnp.max(s, axis=1, keepdims=True)
    p = jnp.exp(s - mx)
    srow = jnp.sum(p, axis=1, keepdims=True)
    m = m_ref[0, 0]
    pm = p * jnp.maximum(m[:_NS, None], m[None, :])
    denom = jnp.sum(pm, axis=1, keepdims=True) + 1e-8 * srow
    ctx = lax.dot_general(pm, v_ref[0, 0], (((1,), (0,)), ((), ())),
                          preferred_element_type=jnp.float32)
    ctx = ctx / denom
    for i in range(_H):
        @pl.when(h == i)
        def _(i=i):
            ctx_ref[:, i * _Dh:(i + 1) * _Dh] = ctx

    @pl.when(h == _H - 1)
    def _():
        out_ref[0] = lax.dot_general(
            ctx_ref[...], wp_ref[0], cdims,
            preferred_element_type=jnp.float32) + bp_ref[0, 0]


def kernel(x, Wqkv, bqkv, Wproj, bproj, count_score_buffer, iteration):
    f32 = jnp.float32
    Wq = Wqkv[0 * _C:1 * _C].reshape(_H, _Dh, _C)
    Wk = Wqkv[1 * _C:2 * _C].reshape(_H, _Dh, _C)
    Wv = Wqkv[2 * _C:3 * _C].reshape(_H, _Dh, _C)
    bq = bqkv[0 * _C:1 * _C].reshape(_H, 1, _Dh)
    bk = bqkv[1 * _C:2 * _C].reshape(_H, 1, _Dh)
    bv = bqkv[2 * _C:3 * _C].reshape(_H, 1, _Dh)
    ucb_expl = jnp.sqrt(jnp.log(jnp.asarray(iteration + 1.0, dtype=f32))
                        / (count_score_buffer[:, 1:] + 1e-06))
    expl = jnp.zeros((_H, _NP), f32).at[:, 1:_N].set(ucb_expl)
    expl = expl.reshape(_H, 1, _NP)
    Wp = Wproj.reshape(1, _C, _C)
    bp = bproj.reshape(1, 1, _C)

    k_shape = jax.ShapeDtypeStruct((_B, _H, _NP, _Dh), f32)
    v_shape = jax.ShapeDtypeStruct((_B, _H, _NP, _Dh), f32)
    q_shape = jax.ShapeDtypeStruct((_B, _H, _NS, _Dh), f32)
    q, k, v, gucb = pl.pallas_call(
        _stats_body,
        grid=(_B, _H),
        in_specs=[
            pl.BlockSpec((1, _NP, _C), lambda b, h: (b, 0, 0)),
            pl.BlockSpec((1, _Dh, _C), lambda b, h: (h, 0, 0)),
            pl.BlockSpec((1, _Dh, _C), lambda b, h: (h, 0, 0)),
            pl.BlockSpec((1, _Dh, _C), lambda b, h: (h, 0, 0)),
            pl.BlockSpec((1, 1, _Dh), lambda b, h: (h, 0, 0)),
            pl.BlockSpec((1, 1, _Dh), lambda b, h: (h, 0, 0)),
            pl.BlockSpec((1, 1, _Dh), lambda b, h: (h, 0, 0)),
            pl.BlockSpec((1, 1, _NP), lambda b, h: (h, 0, 0)),
        ],
        out_specs=[
            pl.BlockSpec((1, 1, _NS, _Dh), lambda b, h: (b, h, 0, 0)),
            pl.BlockSpec((1, 1, _NP, _Dh), lambda b, h: (b, h, 0, 0)),
            pl.BlockSpec((1, 1, _NP, _Dh), lambda b, h: (b, h, 0, 0)),
            pl.BlockSpec((1, 1, _NP), lambda b, h: (b, 0, 0)),
        ],
        out_shape=[q_shape, k_shape, v_shape,
                   jax.ShapeDtypeStruct((_B, 1, _NP), f32)],
        scratch_shapes=[pltpu.VMEM((_H, _NP), f32)],
    )(x, Wq, Wk, Wv, bq, bk, bv, expl)

    mask = pl.kernel(
        _sc_mask_body,
        mesh=plsc.VectorSubcoreMesh(core_axis_name="c", subcore_axis_name="s"),
        compiler_params=pltpu.CompilerParams(needs_layout_passes=False),
        out_type=jax.ShapeDtypeStruct((_B, 1, _NP), f32),
        scratch_types=[
            pltpu.VMEM((_NP,), f32),
            pltpu.VMEM((_NP,), jnp.int32),
            pltpu.VMEM((_NP,), f32),
        ],
    )(gucb)

    out_p = pl.pallas_call(
        _attn_body,
        grid=(_B, _H),
        in_specs=[
            pl.BlockSpec((1, 1, _NS, _Dh), lambda b, h: (b, h, 0, 0)),
            pl.BlockSpec((1, 1, _NP, _Dh), lambda b, h: (b, h, 0, 0)),
            pl.BlockSpec((1, 1, _NP, _Dh), lambda b, h: (b, h, 0, 0)),
            pl.BlockSpec((1, 1, _NP), lambda b, h: (b, 0, 0)),
            pl.BlockSpec((1, _C, _C), lambda b, h: (0, 0, 0)),
            pl.BlockSpec((1, 1, _C), lambda b, h: (0, 0, 0)),
        ],
        out_specs=pl.BlockSpec((1, _NS, _C), lambda b, h: (b, 0, 0)),
        out_shape=jax.ShapeDtypeStruct((_B, _NS, _C), f32),
        scratch_shapes=[pltpu.VMEM((_NS, _C), f32)],
    )(q, k, v, mask, Wp, bp)

    return out_p[:, :_N, :]


# speedup vs baseline: 1.0789x; 1.0333x over previous
import functools

import jax
import jax.numpy as jnp
from jax import lax
from jax.experimental import pallas as pl
from jax.experimental.pallas import tpu as pltpu
from jax.experimental.pallas import tpu_sc as plsc

_B, _N, _C, _H = 8, 577, 768, 12
_Dh = _C // _H
_SCALE = _Dh ** -0.5
_KEEP = max(1, int((_N - 1) * 0.5))
_NP = 640
_LANES = 16
_NCHUNK = _NP // _LANES
_I32_MIN = -2147483648
_I32_MAX = 2147483647
_NEG = -1e30


_NS = 584


_HPB = 2
_HG = _H // _HPB


def _stats_body(x_ref, wq_ref, wk_ref, wv_ref, bq_ref, bk_ref, bv_ref,
                expl_ref, q_ref, k_ref, v_ref, gucb_ref, sc_ref):
    hh = pl.program_id(1)
    xb = x_ref[0]
    cdims = (((1,), (1,)), ((), ()))
    for i in range(_HPB):
        q = lax.dot_general(xb, wq_ref[i], cdims,
                            preferred_element_type=jnp.float32) + bq_ref[i, 0]
        k = lax.dot_general(xb, wk_ref[i], cdims,
                            preferred_element_type=jnp.float32) + bk_ref[i, 0]
        v = lax.dot_general(xb, wv_ref[i], cdims,
                            preferred_element_type=jnp.float32) + bv_ref[i, 0]
        vrow = lax.broadcasted_iota(jnp.int32, (_NP, _Dh), 0)
        v = jnp.where(vrow < _N, v, 0.0)
        q_ref[0, i] = q[:_NS]
        k_ref[0, i] = k
        v_ref[0, i] = v

        s = lax.dot_general(q[:_NS], k, cdims,
                            preferred_element_type=jnp.float32) * _SCALE
        colid = lax.broadcasted_iota(jnp.int32, (_NS, _NP), 1)
        s = jnp.where(colid < _N, s, _NEG)
        mx = jnp.max(s, axis=1, keepdims=True)
        p = jnp.exp(s - mx)
        probs = p / jnp.sum(p, axis=1, keepdims=True)
        rowid = lax.broadcasted_iota(jnp.int32, (_NS, _NP), 0)
        probs = jnp.where(rowid < _N, probs, 0.0)
        patch = jnp.sum(probs, axis=0) / jnp.float32(_N)
        sc = patch + expl_ref[i, 0]
        for j in range(_HG):
            @pl.when(hh == j)
            def _(i=i, j=j):
                sc_ref[j * _HPB + i] = sc

    @pl.when(hh == _HG - 1)
    def _():
        gucb_ref[0, 0] = jnp.sum(sc_ref[...], axis=0) / jnp.float32(_H)


def _sc_mask_body(gucb_hbm, mask_hbm, row_v, keys_v, mask_v):
    wid = lax.axis_index("s") * 2 + lax.axis_index("c")

    @pl.when(wid < _B)
    def _():
        pltpu.sync_copy(gucb_hbm.at[wid, 0], row_v)
        for c in range(_NCHUNK):
            bits = lax.bitcast_convert_type(row_v[pl.ds(c * _LANES, _LANES)],
                                            jnp.int32)
            key = jnp.where(bits >= 0, bits, _I32_MIN - bits)
            lane = lax.iota(jnp.int32, 16) + (c * _LANES)
            valid = (lane >= 1) & (lane <= _N - 1)
            keys_v[pl.ds(c * _LANES, _LANES)] = jnp.where(valid, key, _I32_MIN)

        def count_gt(t):
            acc = jnp.zeros((16,), jnp.int32)
            for c in range(_NCHUNK):
                kc = keys_v[pl.ds(c * _LANES, _LANES)]
                acc = acc + plsc.all_reduce_population_count(kc > t)
            return acc

        def bs_body(_, carry):
            lo, hi = carry
            mid = (lo & hi) + ((lo ^ hi) >> 1)
            take = lo < hi
            p = count_gt(mid) < _KEEP
            lo2 = jnp.where(p, lo, mid + 1)
            hi2 = jnp.where(p, mid, hi)
            return (jnp.where(take, lo2, lo), jnp.where(take, hi2, hi))

        lo, _hi = lax.fori_loop(0, 32, bs_body,
                                (jnp.full((16,), _I32_MIN, jnp.int32),
                                 jnp.full((16,), _I32_MAX, jnp.int32)))
        t = lo
        nties = _KEEP - count_gt(t)

        def cnt_eq_le(m):
            acc = jnp.zeros((16,), jnp.int32)
            for c in range(_NCHUNK):
                kc = keys_v[pl.ds(c * _LANES, _LANES)]
                lane = lax.iota(jnp.int32, 16) + (c * _LANES)
                acc = acc + plsc.all_reduce_population_count(
                    (kc == t) & (lane <= m))
            return acc

        def bs2_body(_, carry):
            lo2, hi2 = carry
            mid = (lo2 + hi2) >> 1
            take = lo2 < hi2
            p = cnt_eq_le(mid) >= nties
            nlo = jnp.where(p, lo2, mid + 1)
            nhi = jnp.where(p, mid, hi2)
            return (jnp.where(take, nlo, lo2), jnp.where(take, nhi, hi2))

        m0, _m1 = lax.fori_loop(0, 10, bs2_body,
                                (jnp.zeros((16,), jnp.int32),
                                 jnp.full((16,), _NP - 1, jnp.int32)))
        mstar = jnp.where(nties > 0, m0, jnp.full((16,), -1, jnp.int32))

        for c in range(_NCHUNK):
            kc = keys_v[pl.ds(c * _LANES, _LANES)]
            lane = lax.iota(jnp.int32, 16) + (c * _LANES)
            keep = (kc > t) | ((kc == t) & (lane <= mstar))
            mvec = jnp.where(keep, 1.0, 0.0)
            if c == 0:
                mvec = jnp.where(lax.iota(jnp.int32, 16) == 0, 1.0, mvec)
            mask_v[pl.ds(c * _LANES, _LANES)] = mvec
        pltpu.sync_copy(mask_v, mask_hbm.at[wid, 0])


def _attn_body(q_ref, k_ref, v_ref, m_ref, wp_ref, bp_ref, out_ref, ctx_ref):
    hh = pl.program_id(1)
    cdims = (((1,), (1,)), ((), ()))
    m = m_ref[0, 0]
    mboth = jnp.maximum(m[:_NS, None], m[None, :])
    for i in range(_HPB):
        s = lax.dot_general(q_ref[0, i], k_ref[0, i], cdims,
                            preferred_element_type=jnp.float32) * _SCALE
        colid = lax.broadcasted_iota(jnp.int32, (_NS, _NP), 1)
        s = jnp.where(colid < _N, s, _NEG)
        mx = jnp.max(s, axis=1, keepdims=True)
        p = jnp.exp(s - mx)
        srow = jnp.sum(p, axis=1, keepdims=True)
        pm = p * mboth
        denom = jnp.sum(pm, axis=1, keepdims=True) + 1e-8 * srow
        ctx = lax.dot_general(pm, v_ref[0, i], (((1,), (0,)), ((), ())),
                              preferred_element_type=jnp.float32)
        ctx = ctx / denom
        for j in range(_HG):
            @pl.when(hh == j)
            def _(i=i, j=j):
                col = (j * _HPB + i) * _Dh
                ctx_ref[:, col:col + _Dh] = ctx

    @pl.when(hh == _HG - 1)
    def _():
        out_ref[0] = lax.dot_general(
            ctx_ref[...], wp_ref[0], cdims,
            preferred_element_type=jnp.float32) + bp_ref[0, 0]


def kernel(x, Wqkv, bqkv, Wproj, bproj, count_score_buffer, iteration):
    f32 = jnp.float32
    Wq = Wqkv[0 * _C:1 * _C].reshape(_H, _Dh, _C)
    Wk = Wqkv[1 * _C:2 * _C].reshape(_H, _Dh, _C)
    Wv = Wqkv[2 * _C:3 * _C].reshape(_H, _Dh, _C)
    bq = bqkv[0 * _C:1 * _C].reshape(_H, 1, _Dh)
    bk = bqkv[1 * _C:2 * _C].reshape(_H, 1, _Dh)
    bv = bqkv[2 * _C:3 * _C].reshape(_H, 1, _Dh)
    ucb_expl = jnp.sqrt(jnp.log(jnp.asarray(iteration + 1.0, dtype=f32))
                        / (count_score_buffer[:, 1:] + 1e-06))
    expl = jnp.zeros((_H, _NP), f32).at[:, 1:_N].set(ucb_expl)
    expl = expl.reshape(_H, 1, _NP)
    Wp = Wproj.reshape(1, _C, _C)
    bp = bproj.reshape(1, 1, _C)

    k_shape = jax.ShapeDtypeStruct((_B, _H, _NP, _Dh), f32)
    v_shape = jax.ShapeDtypeStruct((_B, _H, _NP, _Dh), f32)
    q_shape = jax.ShapeDtypeStruct((_B, _H, _NS, _Dh), f32)
    q, k, v, gucb = pl.pallas_call(
        _stats_body,
        grid=(_B, _HG),
        in_specs=[
            pl.BlockSpec((1, _NP, _C), lambda b, h: (b, 0, 0)),
            pl.BlockSpec((_HPB, _Dh, _C), lambda b, h: (h, 0, 0)),
            pl.BlockSpec((_HPB, _Dh, _C), lambda b, h: (h, 0, 0)),
            pl.BlockSpec((_HPB, _Dh, _C), lambda b, h: (h, 0, 0)),
            pl.BlockSpec((_HPB, 1, _Dh), lambda b, h: (h, 0, 0)),
            pl.BlockSpec((_HPB, 1, _Dh), lambda b, h: (h, 0, 0)),
            pl.BlockSpec((_HPB, 1, _Dh), lambda b, h: (h, 0, 0)),
            pl.BlockSpec((_HPB, 1, _NP), lambda b, h: (h, 0, 0)),
        ],
        out_specs=[
            pl.BlockSpec((1, _HPB, _NS, _Dh), lambda b, h: (b, h, 0, 0)),
            pl.BlockSpec((1, _HPB, _NP, _Dh), lambda b, h: (b, h, 0, 0)),
            pl.BlockSpec((1, _HPB, _NP, _Dh), lambda b, h: (b, h, 0, 0)),
            pl.BlockSpec((1, 1, _NP), lambda b, h: (b, 0, 0)),
        ],
        out_shape=[q_shape, k_shape, v_shape,
                   jax.ShapeDtypeStruct((_B, 1, _NP), f32)],
        scratch_shapes=[pltpu.VMEM((_H, _NP), f32)],
    )(x, Wq, Wk, Wv, bq, bk, bv, expl)

    mask = pl.kernel(
        _sc_mask_body,
        mesh=plsc.VectorSubcoreMesh(core_axis_name="c", subcore_axis_name="s"),
        compiler_params=pltpu.CompilerParams(needs_layout_passes=False),
        out_type=jax.ShapeDtypeStruct((_B, 1, _NP), f32),
        scratch_types=[
            pltpu.VMEM((_NP,), f32),
            pltpu.VMEM((_NP,), jnp.int32),
            pltpu.VMEM((_NP,), f32),
        ],
    )(gucb)

    out_p = pl.pallas_call(
        _attn_body,
        grid=(_B, _HG),
        in_specs=[
            pl.BlockSpec((1, _HPB, _NS, _Dh), lambda b, h: (b, h, 0, 0)),
            pl.BlockSpec((1, _HPB, _NP, _Dh), lambda b, h: (b, h, 0, 0)),
            pl.BlockSpec((1, _HPB, _NP, _Dh), lambda b, h: (b, h, 0, 0)),
            pl.BlockSpec((1, 1, _NP), lambda b, h: (b, 0, 0)),
            pl.BlockSpec((1, _C, _C), lambda b, h: (0, 0, 0)),
            pl.BlockSpec((1, 1, _C), lambda b, h: (0, 0, 0)),
        ],
        out_specs=pl.BlockSpec((1, _NS, _C), lambda b, h: (b, 0, 0)),
        out_shape=jax.ShapeDtypeStruct((_B, _NS, _C), f32),
        scratch_shapes=[pltpu.VMEM((_NS, _C), f32)],
    )(q, k, v, mask, Wp, bp)

    return out_p[:, :_N, :]


# speedup vs baseline: 1.1054x; 1.0245x over previous
import functools

import jax
import jax.numpy as jnp
from jax import lax
from jax.experimental import pallas as pl
from jax.experimental.pallas import tpu as pltpu
from jax.experimental.pallas import tpu_sc as plsc

_B, _N, _C, _H = 8, 577, 768, 12
_Dh = _C // _H
_SCALE = _Dh ** -0.5
_KEEP = max(1, int((_N - 1) * 0.5))
_NP = 640
_LANES = 16
_NCHUNK = _NP // _LANES
_I32_MIN = -2147483648
_I32_MAX = 2147483647
_NEG = -1e30


_NS = 584


_HPB = 3
_HG = _H // _HPB


def _stats_body(x_ref, wq_ref, wk_ref, wv_ref, bq_ref, bk_ref, bv_ref,
                expl_ref, q_ref, k_ref, v_ref, gucb_ref, sc_ref):
    hh = pl.program_id(1)
    xb = x_ref[0]
    cdims = (((1,), (1,)), ((), ()))
    for i in range(_HPB):
        q = lax.dot_general(xb, wq_ref[i], cdims,
                            preferred_element_type=jnp.float32) + bq_ref[i, 0]
        k = lax.dot_general(xb, wk_ref[i], cdims,
                            preferred_element_type=jnp.float32) + bk_ref[i, 0]
        v = lax.dot_general(xb, wv_ref[i], cdims,
                            preferred_element_type=jnp.float32) + bv_ref[i, 0]
        vrow = lax.broadcasted_iota(jnp.int32, (_NP, _Dh), 0)
        v = jnp.where(vrow < _N, v, 0.0)
        q_ref[0, i] = q[:_NS]
        k_ref[0, i] = k
        v_ref[0, i] = v

        s = lax.dot_general(q[:_NS], k, cdims,
                            preferred_element_type=jnp.float32) * _SCALE
        colid = lax.broadcasted_iota(jnp.int32, (_NS, _NP), 1)
        s = jnp.where(colid < _N, s, _NEG)
        mx = jnp.max(s, axis=1, keepdims=True)
        p = jnp.exp(s - mx)
        probs = p / jnp.sum(p, axis=1, keepdims=True)
        rowid = lax.broadcasted_iota(jnp.int32, (_NS, _NP), 0)
        probs = jnp.where(rowid < _N, probs, 0.0)
        patch = jnp.sum(probs, axis=0) / jnp.float32(_N)
        sc = patch + expl_ref[i, 0]
        for j in range(_HG):
            @pl.when(hh == j)
            def _(i=i, j=j):
                sc_ref[j * _HPB + i] = sc

    @pl.when(hh == _HG - 1)
    def _():
        gucb_ref[0, 0] = jnp.sum(sc_ref[...], axis=0) / jnp.float32(_H)


def _sc_mask_body(gucb_hbm, mask_hbm, row_v, keys_v, mask_v):
    wid = lax.axis_index("s") * 2 + lax.axis_index("c")

    @pl.when(wid < _B)
    def _():
        pltpu.sync_copy(gucb_hbm.at[wid, 0], row_v)
        for c in range(_NCHUNK):
            bits = lax.bitcast_convert_type(row_v[pl.ds(c * _LANES, _LANES)],
                                            jnp.int32)
            key = jnp.where(bits >= 0, bits, _I32_MIN - bits)
            lane = lax.iota(jnp.int32, 16) + (c * _LANES)
            valid = (lane >= 1) & (lane <= _N - 1)
            keys_v[pl.ds(c * _LANES, _LANES)] = jnp.where(valid, key, _I32_MIN)

        def count_gt(t):
            acc = jnp.zeros((16,), jnp.int32)
            for c in range(_NCHUNK):
                kc = keys_v[pl.ds(c * _LANES, _LANES)]
                acc = acc + plsc.all_reduce_population_count(kc > t)
            return acc

        def bs_body(_, carry):
            lo, hi = carry
            mid = (lo & hi) + ((lo ^ hi) >> 1)
            take = lo < hi
            p = count_gt(mid) < _KEEP
            lo2 = jnp.where(p, lo, mid + 1)
            hi2 = jnp.where(p, mid, hi)
            return (jnp.where(take, lo2, lo), jnp.where(take, hi2, hi))

        lo, _hi = lax.fori_loop(0, 32, bs_body,
                                (jnp.full((16,), _I32_MIN, jnp.int32),
                                 jnp.full((16,), _I32_MAX, jnp.int32)))
        t = lo
        nties = _KEEP - count_gt(t)

        def cnt_eq_le(m):
            acc = jnp.zeros((16,), jnp.int32)
            for c in range(_NCHUNK):
                kc = keys_v[pl.ds(c * _LANES, _LANES)]
                lane = lax.iota(jnp.int32, 16) + (c * _LANES)
                acc = acc + plsc.all_reduce_population_count(
                    (kc == t) & (lane <= m))
            return acc

        def bs2_body(_, carry):
            lo2, hi2 = carry
            mid = (lo2 + hi2) >> 1
            take = lo2 < hi2
            p = cnt_eq_le(mid) >= nties
            nlo = jnp.where(p, lo2, mid + 1)
            nhi = jnp.where(p, mid, hi2)
            return (jnp.where(take, nlo, lo2), jnp.where(take, nhi, hi2))

        m0, _m1 = lax.fori_loop(0, 10, bs2_body,
                                (jnp.zeros((16,), jnp.int32),
                                 jnp.full((16,), _NP - 1, jnp.int32)))
        mstar = jnp.where(nties > 0, m0, jnp.full((16,), -1, jnp.int32))

        for c in range(_NCHUNK):
            kc = keys_v[pl.ds(c * _LANES, _LANES)]
            lane = lax.iota(jnp.int32, 16) + (c * _LANES)
            keep = (kc > t) | ((kc == t) & (lane <= mstar))
            mvec = jnp.where(keep, 1.0, 0.0)
            if c == 0:
                mvec = jnp.where(lax.iota(jnp.int32, 16) == 0, 1.0, mvec)
            mask_v[pl.ds(c * _LANES, _LANES)] = mvec
        pltpu.sync_copy(mask_v, mask_hbm.at[wid, 0])


def _attn_body(q_ref, k_ref, v_ref, m_ref, wp_ref, bp_ref, out_ref, ctx_ref):
    hh = pl.program_id(1)
    cdims = (((1,), (1,)), ((), ()))
    m = m_ref[0, 0]
    mboth = jnp.maximum(m[:_NS, None], m[None, :])
    for i in range(_HPB):
        s = lax.dot_general(q_ref[0, i], k_ref[0, i], cdims,
                            preferred_element_type=jnp.float32) * _SCALE
        colid = lax.broadcasted_iota(jnp.int32, (_NS, _NP), 1)
        s = jnp.where(colid < _N, s, _NEG)
        mx = jnp.max(s, axis=1, keepdims=True)
        p = jnp.exp(s - mx)
        srow = jnp.sum(p, axis=1, keepdims=True)
        pm = p * mboth
        denom = jnp.sum(pm, axis=1, keepdims=True) + 1e-8 * srow
        ctx = lax.dot_general(pm, v_ref[0, i], (((1,), (0,)), ((), ())),
                              preferred_element_type=jnp.float32)
        ctx = ctx / denom
        for j in range(_HG):
            @pl.when(hh == j)
            def _(i=i, j=j):
                col = (j * _HPB + i) * _Dh
                ctx_ref[:, col:col + _Dh] = ctx

    @pl.when(hh == _HG - 1)
    def _():
        out_ref[0] = lax.dot_general(
            ctx_ref[...], wp_ref[0], cdims,
            preferred_element_type=jnp.float32) + bp_ref[0, 0]


def kernel(x, Wqkv, bqkv, Wproj, bproj, count_score_buffer, iteration):
    f32 = jnp.float32
    Wq = Wqkv[0 * _C:1 * _C].reshape(_H, _Dh, _C)
    Wk = Wqkv[1 * _C:2 * _C].reshape(_H, _Dh, _C)
    Wv = Wqkv[2 * _C:3 * _C].reshape(_H, _Dh, _C)
    bq = bqkv[0 * _C:1 * _C].reshape(_H, 1, _Dh)
    bk = bqkv[1 * _C:2 * _C].reshape(_H, 1, _Dh)
    bv = bqkv[2 * _C:3 * _C].reshape(_H, 1, _Dh)
    ucb_expl = jnp.sqrt(jnp.log(jnp.asarray(iteration + 1.0, dtype=f32))
                        / (count_score_buffer[:, 1:] + 1e-06))
    expl = jnp.zeros((_H, _NP), f32).at[:, 1:_N].set(ucb_expl)
    expl = expl.reshape(_H, 1, _NP)
    Wp = Wproj.reshape(1, _C, _C)
    bp = bproj.reshape(1, 1, _C)

    k_shape = jax.ShapeDtypeStruct((_B, _H, _NP, _Dh), f32)
    v_shape = jax.ShapeDtypeStruct((_B, _H, _NP, _Dh), f32)
    q_shape = jax.ShapeDtypeStruct((_B, _H, _NS, _Dh), f32)
    q, k, v, gucb = pl.pallas_call(
        _stats_body,
        grid=(_B, _HG),
        in_specs=[
            pl.BlockSpec((1, _NP, _C), lambda b, h: (b, 0, 0)),
            pl.BlockSpec((_HPB, _Dh, _C), lambda b, h: (h, 0, 0)),
            pl.BlockSpec((_HPB, _Dh, _C), lambda b, h: (h, 0, 0)),
            pl.BlockSpec((_HPB, _Dh, _C), lambda b, h: (h, 0, 0)),
            pl.BlockSpec((_HPB, 1, _Dh), lambda b, h: (h, 0, 0)),
            pl.BlockSpec((_HPB, 1, _Dh), lambda b, h: (h, 0, 0)),
            pl.BlockSpec((_HPB, 1, _Dh), lambda b, h: (h, 0, 0)),
            pl.BlockSpec((_HPB, 1, _NP), lambda b, h: (h, 0, 0)),
        ],
        out_specs=[
            pl.BlockSpec((1, _HPB, _NS, _Dh), lambda b, h: (b, h, 0, 0)),
            pl.BlockSpec((1, _HPB, _NP, _Dh), lambda b, h: (b, h, 0, 0)),
            pl.BlockSpec((1, _HPB, _NP, _Dh), lambda b, h: (b, h, 0, 0)),
            pl.BlockSpec((1, 1, _NP), lambda b, h: (b, 0, 0)),
        ],
        out_shape=[q_shape, k_shape, v_shape,
                   jax.ShapeDtypeStruct((_B, 1, _NP), f32)],
        scratch_shapes=[pltpu.VMEM((_H, _NP), f32)],
    )(x, Wq, Wk, Wv, bq, bk, bv, expl)

    mask = pl.kernel(
        _sc_mask_body,
        mesh=plsc.VectorSubcoreMesh(core_axis_name="c", subcore_axis_name="s"),
        compiler_params=pltpu.CompilerParams(needs_layout_passes=False),
        out_type=jax.ShapeDtypeStruct((_B, 1, _NP), f32),
        scratch_types=[
            pltpu.VMEM((_NP,), f32),
            pltpu.VMEM((_NP,), jnp.int32),
            pltpu.VMEM((_NP,), f32),
        ],
    )(gucb)

    out_p = pl.pallas_call(
        _attn_body,
        grid=(_B, _HG),
        in_specs=[
            pl.BlockSpec((1, _HPB, _NS, _Dh), lambda b, h: (b, h, 0, 0)),
            pl.BlockSpec((1, _HPB, _NP, _Dh), lambda b, h: (b, h, 0, 0)),
            pl.BlockSpec((1, _HPB, _NP, _Dh), lambda b, h: (b, h, 0, 0)),
            pl.BlockSpec((1, 1, _NP), lambda b, h: (b, 0, 0)),
            pl.BlockSpec((1, _C, _C), lambda b, h: (0, 0, 0)),
            pl.BlockSpec((1, 1, _C), lambda b, h: (0, 0, 0)),
        ],
        out_specs=pl.BlockSpec((1, _NS, _C), lambda b, h: (b, 0, 0)),
        out_shape=jax.ShapeDtypeStruct((_B, _NS, _C), f32),
        scratch_shapes=[pltpu.VMEM((_NS, _C), f32)],
    )(q, k, v, mask, Wp, bp)

    return out_p[:, :_N, :]
